# baseline (speedup 1.0000x reference)
import functools

import jax
import jax.numpy as jnp
from jax.experimental import pallas as pl

NA_E = 128
NA_O = 384
PPS = 8


def _dense(pm, fb, shifts, sv, mask_e, oh, *, T, P, W, EV_W):
    f32 = jnp.float32
    s_e = sv[0, 0]
    s_o = sv[0, 1]
    h_e = 0.5 / (s_e * s_e)
    h_o = 0.5 / (s_o * s_o)
    h = jnp.where(mask_e, h_e, h_o)
    hn = jnp.where(mask_e, -h_e, -h_o)
    h2 = jnp.where(mask_e, h_e + h_e, h_o + h_o)
    u2 = jnp.where(mask_e, jnp.exp(-(h_e + h_e)), jnp.exp(-(h_o + h_o)))
    ninv = jnp.where(mask_e, -(h_e + h_e), -(h_o + h_o))
    shift_exp = jnp.dot(shifts, oh, preferred_element_type=f32)

    lp = None
    tars = []
    for d in range(3):
        f_d = fb[d:d + 1]
        pm_d = pm[d:d + 1]
        ys = [f_d[:, :NA_E] - pm_d[:, p * NA_E:(p + 1) * NA_E]
              for p in range(P)]
        ys += [f_d[:, NA_E:] - pm_d[:, EV_W + p * NA_O:EV_W + (p + 1) * NA_O]
               for p in range(P)]
        y = jnp.concatenate(ys, axis=1)
        z = y - shift_exp[d * T:(d + 1) * T]
        w0 = z - jnp.round(z)
        q = (w0 * w0) * h
        t1 = w0 * h2
        ep1 = jnp.exp(hn - t1)
        em1 = jnp.exp(hn + t1)
        cp = ep1 * u2
        cm = em1 * u2
        ep2 = ep1 * cp
        em2 = em1 * cm
        ep3 = ep2 * (cp * u2)
        em3 = em2 * (cm * u2)
        se = 1.0 + (ep1 + em1) + (ep2 + em2) + (ep3 + em3)
        sj = (ep1 - em1) + 2.0 * (ep2 - em2) + 3.0 * (ep3 - em3)
        lpd = jnp.log(se) - q
        lp = lpd if lp is None else lp + lpd
        tars.append(ninv * (w0 + sj / se))
    return lp, tars


def _tail(lp, tars, oh, pred, sn, *, T, P, W, EV_W):
    f32 = jnp.float32
    rlp = jax.lax.dot_general(lp, oh, (((1,), (1,)), ((), ())),
                              preferred_element_type=f32)
    lane16 = jax.lax.broadcasted_iota(jnp.int32, (1, 2 * P), 1)
    even_half = lane16 < P
    NEGBIG = jnp.float32(-1e30)
    m_e = jnp.max(jnp.where(even_half, rlp, NEGBIG))
    m_o = jnp.max(jnp.where(even_half, NEGBIG, rlp))
    e = jnp.exp(rlp - jnp.where(even_half, m_e, m_o))
    z_e = jnp.sum(jnp.where(even_half, e, 0.0))
    z_o = jnp.sum(jnp.where(even_half, 0.0, e))
    w = e / jnp.where(even_half, z_e, z_o)
    w_exp = jnp.dot(w, oh, preferred_element_type=f32)

    rows = []
    for d in range(3):
        prod = tars[d] * w_exp
        acc_e = prod[:, :NA_E]
        for p in range(1, P):
            acc_e = acc_e + prod[:, p * NA_E:(p + 1) * NA_E]
        acc_o = prod[:, EV_W:EV_W + NA_O]
        for p in range(1, P):
            base = EV_W + p * NA_O
            acc_o = acc_o + prod[:, base:base + NA_O]
        pair = jnp.concatenate([acc_e, acc_o], axis=1)
        rows.append(jnp.sum(pair, axis=0, keepdims=True))
    tar3 = jnp.concatenate(rows, axis=0)
    tar3 = tar3 / jnp.sqrt(sn)
    dlt = pred - tar3
    return jnp.sum(dlt * dlt)


def _fused(perm_ref, frac_ref, shifts_ref, sig_ref, pred_ref, sn_ref, loss_ref,
           *, T, P, W, EV_W, NSTEP, NTOT):
    i = pl.program_id(0)
    PAIR_A = NA_E + NA_O
    lane = jax.lax.broadcasted_iota(jnp.int32, (1, W), 1)
    seg = jnp.where(lane < EV_W, lane // NA_E, P + (lane - EV_W) // NA_O)
    rows16 = jax.lax.broadcasted_iota(jnp.int32, (2 * P, W), 0)
    oh = (rows16 == seg).astype(jnp.float32)

    kw = dict(T=T, P=P, W=W, EV_W=EV_W)

    def run_dense(pp):
        lo = pp * W
        la = pp * PAIR_A
        return _dense(perm_ref[:, lo:lo + W], frac_ref[:, la:la + PAIR_A],
                      shifts_ref[pp], sig_ref[pp], lane < EV_W, oh, **kw)

    def run_tail(pp, state):
        la = pp * PAIR_A
        lp, tars = state
        return _tail(lp, tars, oh, pred_ref[:, la:la + PAIR_A],
                     sn_ref[:, la:la + PAIR_A], **kw)

    part = None
    prev = run_dense(0)
    for pp in range(1, PPS):
        nxt = run_dense(pp)
        pt = run_tail(pp - 1, prev)
        part = pt if part is None else part + pt
        prev = nxt
    part = part + run_tail(PPS - 1, prev)
    part = part.reshape(1, 1)
    cur = jnp.where(i == 0, 0.0, loss_ref[...])
    tot = cur + part
    loss_ref[...] = jnp.where(i == NSTEP - 1, tot / float(NTOT * 3), tot)


def kernel(frac_coords_t, permuted_frac_coords, sigmas, sigmas_norm_per_atom,
           pred_score_x, random_shifts, num_atoms):
    N = frac_coords_t.shape[0]
    P = permuted_frac_coords.shape[0] // N
    B = sigmas.shape[0]
    T = random_shifts.shape[0]
    NPAIR = B // 2
    NSTEP = NPAIR // PPS
    PAIR_A = NA_E + NA_O
    W = P * PAIR_A
    EV_W = P * NA_E

    f32 = jnp.float32
    permT = permuted_frac_coords.T
    fracT = frac_coords_t.T
    shifts3 = (random_shifts.transpose(2, 0, 1)
               .reshape(3 * T, NPAIR, 2 * P)
               .transpose(1, 0, 2))
    sig3 = sigmas.reshape(NPAIR, 1, 2)
    predT = pred_score_x.T
    snT = sigmas_norm_per_atom.reshape(1, N)

    body = functools.partial(_fused, T=T, P=P, W=W, EV_W=EV_W, NSTEP=NSTEP,
                             NTOT=N)
    loss = pl.pallas_call(
        body,
        grid=(NSTEP,),
        in_specs=[
            pl.BlockSpec((3, PPS * W), lambda i: (0, i)),
            pl.BlockSpec((3, PPS * PAIR_A), lambda i: (0, i)),
            pl.BlockSpec((PPS, 3 * T, 2 * P), lambda i: (i, 0, 0)),
            pl.BlockSpec((PPS, 1, 2), lambda i: (i, 0, 0)),
            pl.BlockSpec((3, PPS * PAIR_A), lambda i: (0, i)),
            pl.BlockSpec((1, PPS * PAIR_A), lambda i: (0, i)),
        ],
        out_specs=pl.BlockSpec((1, 1), lambda i: (0, 0)),
        out_shape=jax.ShapeDtypeStruct((1, 1), f32),
    )(permT, fracT, shifts3, sig3, predT, snT)
    return loss[0, 0]


# speedup vs baseline: 1.0564x; 1.0564x over previous
import functools

import jax
import jax.numpy as jnp
from jax.experimental import pallas as pl

NA_E = 128
NA_O = 384
PPS = 8


def _dense(pm, fb, shifts, sv, mask_e, oh, *, T, P, W, EV_W):
    f32 = jnp.float32
    s_e = sv[0, 0]
    s_o = sv[0, 1]
    h_e = 0.5 / (s_e * s_e)
    h_o = 0.5 / (s_o * s_o)
    h = jnp.where(mask_e, h_e, h_o)
    hn = jnp.where(mask_e, -h_e, -h_o)
    h2 = jnp.where(mask_e, h_e + h_e, h_o + h_o)
    u2 = jnp.where(mask_e, jnp.exp(-(h_e + h_e)), jnp.exp(-(h_o + h_o)))
    ninv = jnp.where(mask_e, -(h_e + h_e), -(h_o + h_o))
    shift_exp = jnp.dot(shifts, oh, preferred_element_type=f32)

    lp = None
    tars = []
    for d in range(3):
        f_d = fb[d:d + 1]
        pm_d = pm[d:d + 1]
        ys = [f_d[:, :NA_E] - pm_d[:, p * NA_E:(p + 1) * NA_E]
              for p in range(P)]
        ys += [f_d[:, NA_E:] - pm_d[:, EV_W + p * NA_O:EV_W + (p + 1) * NA_O]
               for p in range(P)]
        y = jnp.concatenate(ys, axis=1)
        z = y - shift_exp[d * T:(d + 1) * T]
        w0 = z - jnp.round(z)
        q = (w0 * w0) * h
        t1 = w0 * h2
        ep1 = jnp.exp(hn - t1)
        em1 = jnp.exp(hn + t1)
        cp = ep1 * u2
        cm = em1 * u2
        ep2 = ep1 * cp
        em2 = em1 * cm
        ep3 = ep2 * (cp * u2)
        em3 = em2 * (cm * u2)
        se = 1.0 + (ep1 + em1) + (ep2 + em2) + (ep3 + em3)
        sj = (ep1 - em1) + 2.0 * (ep2 - em2) + 3.0 * (ep3 - em3)
        lpd = jnp.log(se) - q
        lp = lpd if lp is None else lp + lpd
        tars.append(ninv * (w0 + sj / se))
    return lp, tars


def _tail(lp, tars, oh, pred, sn, *, T, P, W, EV_W):
    f32 = jnp.float32
    rlp = jax.lax.dot_general(lp, oh, (((1,), (1,)), ((), ())),
                              preferred_element_type=f32)
    lane16 = jax.lax.broadcasted_iota(jnp.int32, (1, 2 * P), 1)
    even_half = lane16 < P
    NEGBIG = jnp.float32(-1e30)
    m_e = jnp.max(jnp.where(even_half, rlp, NEGBIG))
    m_o = jnp.max(jnp.where(even_half, NEGBIG, rlp))
    e = jnp.exp(rlp - jnp.where(even_half, m_e, m_o))
    z_e = jnp.sum(jnp.where(even_half, e, 0.0))
    z_o = jnp.sum(jnp.where(even_half, 0.0, e))
    w = e / jnp.where(even_half, z_e, z_o)
    w_exp = jnp.dot(w, oh, preferred_element_type=f32)

    rows = []
    for d in range(3):
        prod = tars[d] * w_exp
        acc_e = prod[:, :NA_E]
        for p in range(1, P):
            acc_e = acc_e + prod[:, p * NA_E:(p + 1) * NA_E]
        acc_o = prod[:, EV_W:EV_W + NA_O]
        for p in range(1, P):
            base = EV_W + p * NA_O
            acc_o = acc_o + prod[:, base:base + NA_O]
        pair = jnp.concatenate([acc_e, acc_o], axis=1)
        rows.append(jnp.sum(pair, axis=0, keepdims=True))
    tar3 = jnp.concatenate(rows, axis=0)
    tar3 = tar3 / jnp.sqrt(sn)
    dlt = pred - tar3
    return jnp.sum(dlt * dlt)


def _fused(perm_ref, frac_ref, shifts_ref, sig_ref, pred_ref, sn_ref, loss_ref,
           *, T, P, W, EV_W, NSTEP, NTOT):
    i = pl.program_id(0)
    PAIR_A = NA_E + NA_O
    lane = jax.lax.broadcasted_iota(jnp.int32, (1, W), 1)
    seg = jnp.where(lane < EV_W, lane // NA_E, P + (lane - EV_W) // NA_O)
    rows16 = jax.lax.broadcasted_iota(jnp.int32, (2 * P, W), 0)
    oh = (rows16 == seg).astype(jnp.float32)

    kw = dict(T=T, P=P, W=W, EV_W=EV_W)

    def run_dense(pp):
        lo = pp * W
        la = pp * PAIR_A
        return _dense(perm_ref[:, lo:lo + W], frac_ref[:, la:la + PAIR_A],
                      shifts_ref[pp], sig_ref[pp], lane < EV_W, oh, **kw)

    def run_tail(pp, state):
        la = pp * PAIR_A
        lp, tars = state
        return _tail(lp, tars, oh, pred_ref[:, la:la + PAIR_A],
                     sn_ref[:, la:la + PAIR_A], **kw)

    states = [run_dense(pp) for pp in range(PPS)]
    part = None
    for pp in range(PPS):
        pt = run_tail(pp, states[pp])
        part = pt if part is None else part + pt
    part = part.reshape(1, 1)
    if NSTEP == 1:
        loss_ref[...] = part / float(NTOT * 3)
    else:
        cur = jnp.where(i == 0, 0.0, loss_ref[...])
        tot = cur + part
        loss_ref[...] = jnp.where(i == NSTEP - 1, tot / float(NTOT * 3), tot)


def kernel(frac_coords_t, permuted_frac_coords, sigmas, sigmas_norm_per_atom,
           pred_score_x, random_shifts, num_atoms):
    N = frac_coords_t.shape[0]
    P = permuted_frac_coords.shape[0] // N
    B = sigmas.shape[0]
    T = random_shifts.shape[0]
    NPAIR = B // 2
    NSTEP = NPAIR // PPS
    PAIR_A = NA_E + NA_O
    W = P * PAIR_A
    EV_W = P * NA_E

    f32 = jnp.float32
    permT = permuted_frac_coords.T
    fracT = frac_coords_t.T
    shifts3 = (random_shifts.transpose(2, 0, 1)
               .reshape(3 * T, NPAIR, 2 * P)
               .transpose(1, 0, 2))
    sig3 = sigmas.reshape(NPAIR, 1, 2)
    predT = pred_score_x.T
    snT = sigmas_norm_per_atom.reshape(1, N)

    body = functools.partial(_fused, T=T, P=P, W=W, EV_W=EV_W, NSTEP=NSTEP,
                             NTOT=N)
    loss = pl.pallas_call(
        body,
        grid=(NSTEP,),
        in_specs=[
            pl.BlockSpec((3, PPS * W), lambda i: (0, i)),
            pl.BlockSpec((3, PPS * PAIR_A), lambda i: (0, i)),
            pl.BlockSpec((PPS, 3 * T, 2 * P), lambda i: (i, 0, 0)),
            pl.BlockSpec((PPS, 1, 2), lambda i: (i, 0, 0)),
            pl.BlockSpec((3, PPS * PAIR_A), lambda i: (0, i)),
            pl.BlockSpec((1, PPS * PAIR_A), lambda i: (0, i)),
        ],
        out_specs=pl.BlockSpec((1, 1), lambda i: (0, 0)),
        out_shape=jax.ShapeDtypeStruct((1, 1), f32),
    )(permT, fracT, shifts3, sig3, predT, snT)
    return loss[0, 0]


# speedup vs baseline: 1.1146x; 1.0550x over previous
import functools

import jax
import jax.numpy as jnp
from jax.experimental import pallas as pl

NA_E = 128
NA_O = 384
PPS = 8


def _dense(pm, fb, shifts, sv, mask_e, oh, *, T, P, W, EV_W):
    f32 = jnp.float32
    s_e = sv[0, 0]
    s_o = sv[0, 1]
    h_e = 0.5 / (s_e * s_e)
    h_o = 0.5 / (s_o * s_o)
    h = jnp.where(mask_e, h_e, h_o)
    hn = jnp.where(mask_e, -h_e, -h_o)
    h2 = jnp.where(mask_e, h_e + h_e, h_o + h_o)
    u2 = jnp.where(mask_e, jnp.exp(-(h_e + h_e)), jnp.exp(-(h_o + h_o)))
    ninv = jnp.where(mask_e, -(h_e + h_e), -(h_o + h_o))
    shift_exp = jnp.dot(shifts, oh, preferred_element_type=f32)

    lp = None
    tars = []
    for d in range(3):
        f_d = fb[d:d + 1]
        yy = (pm[d:d + 1] + shift_exp[d * T:(d + 1) * T]) + 1.0
        yw = yy - jnp.floor(yy)
        xs = [f_d[:, :NA_E] - yw[:, p * NA_E:(p + 1) * NA_E]
              for p in range(P)]
        xs += [f_d[:, NA_E:] - yw[:, EV_W + p * NA_O:EV_W + (p + 1) * NA_O]
               for p in range(P)]
        x = jnp.concatenate(xs, axis=1)
        w0 = x - jnp.round(x)
        q = (w0 * w0) * h
        t1 = w0 * h2
        ep1 = jnp.exp(hn - t1)
        em1 = jnp.exp(hn + t1)
        cp = ep1 * u2
        cm = em1 * u2
        ep2 = ep1 * cp
        em2 = em1 * cm
        ep3 = ep2 * (cp * u2)
        em3 = em2 * (cm * u2)
        se = 1.0 + (ep1 + em1) + (ep2 + em2) + (ep3 + em3)
        sj = (ep1 - em1) + 2.0 * (ep2 - em2) + 3.0 * (ep3 - em3)
        lpd = jnp.log(se) - q
        lp = lpd if lp is None else lp + lpd
        tars.append(ninv * (w0 + sj / se))
    return lp, tars


def _tail(lp, tars, oh, pred, sn, *, T, P, W, EV_W):
    f32 = jnp.float32
    rlp = jax.lax.dot_general(lp, oh, (((1,), (1,)), ((), ())),
                              preferred_element_type=f32)
    lane16 = jax.lax.broadcasted_iota(jnp.int32, (1, 2 * P), 1)
    even_half = lane16 < P
    NEGBIG = jnp.float32(-1e30)
    m_e = jnp.max(jnp.where(even_half, rlp, NEGBIG))
    m_o = jnp.max(jnp.where(even_half, NEGBIG, rlp))
    e = jnp.exp(rlp - jnp.where(even_half, m_e, m_o))
    z_e = jnp.sum(jnp.where(even_half, e, 0.0))
    z_o = jnp.sum(jnp.where(even_half, 0.0, e))
    w = e / jnp.where(even_half, z_e, z_o)
    w_exp = jnp.dot(w, oh, preferred_element_type=f32)

    rows = []
    for d in range(3):
        prod = tars[d] * w_exp
        acc_e = prod[:, :NA_E]
        for p in range(1, P):
            acc_e = acc_e + prod[:, p * NA_E:(p + 1) * NA_E]
        acc_o = prod[:, EV_W:EV_W + NA_O]
        for p in range(1, P):
            base = EV_W + p * NA_O
            acc_o = acc_o + prod[:, base:base + NA_O]
        pair = jnp.concatenate([acc_e, acc_o], axis=1)
        rows.append(jnp.sum(pair, axis=0, keepdims=True))
    tar3 = jnp.concatenate(rows, axis=0)
    tar3 = tar3 / jnp.sqrt(sn)
    dlt = pred - tar3
    return jnp.sum(dlt * dlt)


def _fused(perm_ref, frac_ref, shifts_ref, sig_ref, pred_ref, sn_ref, loss_ref,
           *, T, P, W, EV_W, NSTEP, NTOT):
    i = pl.program_id(0)
    PAIR_A = NA_E + NA_O
    lane = jax.lax.broadcasted_iota(jnp.int32, (1, W), 1)
    seg = jnp.where(lane < EV_W, lane // NA_E, P + (lane - EV_W) // NA_O)
    rows16 = jax.lax.broadcasted_iota(jnp.int32, (2 * P, W), 0)
    oh = (rows16 == seg).astype(jnp.float32)

    kw = dict(T=T, P=P, W=W, EV_W=EV_W)

    def run_dense(pp):
        lo = pp * W
        la = pp * PAIR_A
        return _dense(perm_ref[:, lo:lo + W], frac_ref[:, la:la + PAIR_A],
                      shifts_ref[pp], sig_ref[pp], lane < EV_W, oh, **kw)

    def run_tail(pp, state):
        la = pp * PAIR_A
        lp, tars = state
        return _tail(lp, tars, oh, pred_ref[:, la:la + PAIR_A],
                     sn_ref[:, la:la + PAIR_A], **kw)

    states = [run_dense(pp) for pp in range(PPS)]
    part = None
    for pp in range(PPS):
        pt = run_tail(pp, states[pp])
        part = pt if part is None else part + pt
    part = part.reshape(1, 1)
    if NSTEP == 1:
        loss_ref[...] = part / float(NTOT * 3)
    else:
        cur = jnp.where(i == 0, 0.0, loss_ref[...])
        tot = cur + part
        loss_ref[...] = jnp.where(i == NSTEP - 1, tot / float(NTOT * 3), tot)


def kernel(frac_coords_t, permuted_frac_coords, sigmas, sigmas_norm_per_atom,
           pred_score_x, random_shifts, num_atoms):
    N = frac_coords_t.shape[0]
    P = permuted_frac_coords.shape[0] // N
    B = sigmas.shape[0]
    T = random_shifts.shape[0]
    NPAIR = B // 2
    NSTEP = NPAIR // PPS
    PAIR_A = NA_E + NA_O
    W = P * PAIR_A
    EV_W = P * NA_E

    f32 = jnp.float32
    permT = permuted_frac_coords.T
    fracT = frac_coords_t.T
    shifts3 = (random_shifts.transpose(2, 0, 1)
               .reshape(3 * T, NPAIR, 2 * P)
               .transpose(1, 0, 2))
    sig3 = sigmas.reshape(NPAIR, 1, 2)
    predT = pred_score_x.T
    snT = sigmas_norm_per_atom.reshape(1, N)

    body = functools.partial(_fused, T=T, P=P, W=W, EV_W=EV_W, NSTEP=NSTEP,
                             NTOT=N)
    loss = pl.pallas_call(
        body,
        grid=(NSTEP,),
        in_specs=[
            pl.BlockSpec((3, PPS * W), lambda i: (0, i)),
            pl.BlockSpec((3, PPS * PAIR_A), lambda i: (0, i)),
            pl.BlockSpec((PPS, 3 * T, 2 * P), lambda i: (i, 0, 0)),
            pl.BlockSpec((PPS, 1, 2), lambda i: (i, 0, 0)),
            pl.BlockSpec((3, PPS * PAIR_A), lambda i: (0, i)),
            pl.BlockSpec((1, PPS * PAIR_A), lambda i: (0, i)),
        ],
        out_specs=pl.BlockSpec((1, 1), lambda i: (0, 0)),
        out_shape=jax.ShapeDtypeStruct((1, 1), f32),
    )(permT, fracT, shifts3, sig3, predT, snT)
    return loss[0, 0]
